# baseline (speedup 1.0000x reference)
import jax
import jax.numpy as jnp
from jax import lax
from jax.experimental import pallas as pl
from jax.experimental.pallas import tpu as pltpu
from jax.experimental.pallas import tpu_sc as plsc

_B = 16384
_D = 128
_NC = 2
_NS = 16
_NW = _NC * _NS
_BPW = _B // _NW
_CH = 128
_NCHUNK = _BPW // _CH


def _gmf_body(users_hbm, items_hbm, utab_hbm, itab_hbm, w_hbm, binit_hbm,
              out_hbm, uidx0, uidx1, iidx0, iidx1, urows0, urows1,
              irows0, irows1, w_v, binit_v, outc_v,
              sem0, sem1):
    cid = lax.axis_index("c")
    sid = lax.axis_index("s")
    wid = sid * _NC + cid
    base = wid * _BPW

    pltpu.sync_copy(w_hbm, w_v)
    pltpu.sync_copy(binit_hbm, binit_v)
    b_init = binit_v[...]
    w_regs = [w_v[pl.ds(j * 16, 16)] for j in range(8)]

    lane = lax.iota(jnp.int32, 16)
    rot8 = (lane + 8) & 15
    rot4 = (lane & 8) | ((lane + 4) & 7)
    rot2 = (lane & 12) | ((lane + 2) & 3)
    rot1 = (lane & 14) | ((lane + 1) & 1)
    bitrev = (((lane & 1) << 3) | ((lane & 2) << 1)
              | ((lane & 4) >> 1) | ((lane & 8) >> 3))
    m8 = lane < 8
    m4 = (lane & 4) == 0
    m2 = (lane & 2) == 0
    m1 = (lane & 1) == 0

    _dnums = lax.GatherDimensionNumbers(
        offset_dims=(), collapsed_slice_dims=(0,), start_index_map=(0,))

    def _perm(x, idx):
        return lax.gather(x, idx[:, None], _dnums, (1,),
                          mode=lax.GatherScatterMode.PROMISE_IN_BOUNDS)

    ubufs = (urows0, urows1)
    ibufs = (irows0, irows1)
    uidxs = (uidx0, uidx1)
    iidxs = (iidx0, iidx1)
    sems = (sem0, sem1)
    pending = [None, None]

    def start(c):
        k = c % 2
        cbase = base + c * _CH
        pltpu.sync_copy(users_hbm.at[pl.ds(cbase, _CH)], uidxs[k])
        pltpu.sync_copy(items_hbm.at[pl.ds(cbase, _CH)], iidxs[k])
        cu = pltpu.async_copy(utab_hbm.at[uidxs[k]], ubufs[k], sems[k])
        ci = pltpu.async_copy(itab_hbm.at[iidxs[k]], ibufs[k], sems[k])
        pending[k] = (cu, ci)

    start(0)
    for c in range(_NCHUNK):
        if c + 1 < _NCHUNK:
            start(c + 1)
        k = c % 2
        cu, ci = pending[k]
        cu.wait()
        ci.wait()
        urows_v = ubufs[k]
        irows_v = ibufs[k]
        cbase = base + c * _CH

        def group_body(g, carry):
            rbase = g * 16
            accs = []
            for rr in range(16):
                r = rbase + rr
                acc = b_init
                for j in range(8):
                    acc = acc + (urows_v[r, pl.ds(j * 16, 16)]
                                 * irows_v[r, pl.ds(j * 16, 16)]
                                 * w_regs[j])
                accs.append(acc)
            p = [jnp.where(m8,
                           accs[2 * i] + _perm(accs[2 * i], rot8),
                           accs[2 * i + 1] + _perm(accs[2 * i + 1], rot8))
                 for i in range(8)]
            q = [pp + _perm(pp, rot4) for pp in p]
            s = [jnp.where(m4, q[2 * j], q[2 * j + 1]) for j in range(4)]
            t = [ss + _perm(ss, rot2) for ss in s]
            u = [jnp.where(m2, t[2 * k], t[2 * k + 1]) for k in range(2)]
            v = [uu + _perm(uu, rot1) for uu in u]
            f0 = jnp.where(m1, v[0], v[1])
            outc_v[pl.ds(rbase, 16)] = _perm(f0, bitrev)
            return carry

        lax.fori_loop(0, _CH // 16, group_body, 0)
        pltpu.sync_copy(outc_v, out_hbm.at[pl.ds(cbase, _CH)])


def kernel(users, items, user_table, item_table, W_beta, b_beta):
    users_i = users.astype(jnp.int32)
    items_i = items.astype(jnp.int32)
    w = W_beta.reshape(_D)
    binit = jnp.pad(b_beta.reshape(1), (0, 15))

    mesh = plsc.VectorSubcoreMesh(core_axis_name="c", subcore_axis_name="s")
    f = pl.kernel(
        _gmf_body,
        mesh=mesh,
        out_type=jax.ShapeDtypeStruct((_B,), jnp.float32),
        scratch_types=[
            pltpu.VMEM((_CH,), jnp.int32),
            pltpu.VMEM((_CH,), jnp.int32),
            pltpu.VMEM((_CH,), jnp.int32),
            pltpu.VMEM((_CH,), jnp.int32),
            pltpu.VMEM((_CH, _D), jnp.float32),
            pltpu.VMEM((_CH, _D), jnp.float32),
            pltpu.VMEM((_CH, _D), jnp.float32),
            pltpu.VMEM((_CH, _D), jnp.float32),
            pltpu.VMEM((_D,), jnp.float32),
            pltpu.VMEM((16,), jnp.float32),
            pltpu.VMEM((_CH,), jnp.float32),
            pltpu.SemaphoreType.DMA,
            pltpu.SemaphoreType.DMA,
        ],
    )
    out = f(users_i, items_i, user_table, item_table, w, binit)
    return out.reshape(_B, 1)


# speedup vs baseline: 1.2489x; 1.2489x over previous
import jax
import jax.numpy as jnp
from jax import lax
from jax.experimental import pallas as pl
from jax.experimental.pallas import tpu as pltpu
from jax.experimental.pallas import tpu_sc as plsc

_B = 16384
_D = 128
_NC = 2
_NS = 16
_NW = _NC * _NS
_BPW = _B // _NW
_CH = 64
_NCHUNK = _BPW // _CH
_NBUF = 4


def _gmf_body(users_hbm, items_hbm, utab_hbm, itab_hbm, w_hbm, binit_hbm,
              out_hbm, uidx_v, iidx_v, ub0, ub1, ub2, ub3, ib0, ib1, ib2,
              ib3, w_v, binit_v, qbuf_v, outc_v, sem0, sem1, sem2, sem3):
    cid = lax.axis_index("c")
    sid = lax.axis_index("s")
    wid = sid * _NC + cid
    base = wid * _BPW

    pltpu.sync_copy(w_hbm, w_v)
    pltpu.sync_copy(binit_hbm, binit_v)
    pltpu.sync_copy(users_hbm.at[pl.ds(base, _BPW)], uidx_v)
    pltpu.sync_copy(items_hbm.at[pl.ds(base, _BPW)], iidx_v)
    b_init = binit_v[...]
    w_regs = [w_v[pl.ds(j * 16, 16)] for j in range(8)]

    lane = lax.iota(jnp.int32, 16)
    rot8 = (lane + 8) & 15
    rot4 = (lane & 8) | ((lane + 4) & 7)
    rot2 = (lane & 12) | ((lane + 2) & 3)
    rot1 = (lane & 14) | ((lane + 1) & 1)
    bitrev = (((lane & 1) << 3) | ((lane & 2) << 1)
              | ((lane & 4) >> 1) | ((lane & 8) >> 3))
    m8 = lane < 8
    m4 = (lane & 4) == 0
    m2 = (lane & 2) == 0
    m1 = (lane & 1) == 0

    _dnums = lax.GatherDimensionNumbers(
        offset_dims=(), collapsed_slice_dims=(0,), start_index_map=(0,))

    def _perm(x, idx):
        return lax.gather(x, idx[:, None], _dnums, (1,),
                          mode=lax.GatherScatterMode.PROMISE_IN_BOUNDS)

    ubufs = (ub0, ub1, ub2, ub3)
    ibufs = (ib0, ib1, ib2, ib3)
    sems = (sem0, sem1, sem2, sem3)
    pending = [None] * _NBUF

    def start(c):
        k = c % _NBUF
        cu = pltpu.async_copy(
            utab_hbm.at[uidx_v.at[pl.ds(c * _CH, _CH)]], ubufs[k], sems[k])
        ci = pltpu.async_copy(
            itab_hbm.at[iidx_v.at[pl.ds(c * _CH, _CH)]], ibufs[k], sems[k])
        pending[k] = (cu, ci)

    for c in range(_NBUF - 1):
        start(c)

    for c in range(_NCHUNK):
        if c + _NBUF - 1 < _NCHUNK:
            start(c + _NBUF - 1)
        k = c % _NBUF
        cu, ci = pending[k]
        cu.wait()
        ci.wait()
        urows_v = ubufs[k]
        irows_v = ibufs[k]

        def row_acc(r):
            acc = b_init
            for j in range(8):
                acc = acc + (urows_v[r, pl.ds(j * 16, 16)]
                             * irows_v[r, pl.ds(j * 16, 16)]
                             * w_regs[j])
            return acc

        def quad_body(i, carry):
            a = row_acc(4 * i)
            b = row_acc(4 * i + 1)
            cc = row_acc(4 * i + 2)
            d = row_acc(4 * i + 3)
            p1 = jnp.where(m8, a + _perm(a, rot8), b + _perm(b, rot8))
            p2 = jnp.where(m8, cc + _perm(cc, rot8), d + _perm(d, rot8))
            q1 = p1 + _perm(p1, rot4)
            q2 = p2 + _perm(p2, rot4)
            s = jnp.where(m4, q1, q2)
            qbuf_v[pl.ds(i * 16, 16)] = s + _perm(s, rot2)
            return carry

        lax.fori_loop(0, _CH // 4, quad_body, 0)

        def merge_body(g, carry):
            t0 = qbuf_v[pl.ds((4 * g) * 16, 16)]
            t1 = qbuf_v[pl.ds((4 * g + 1) * 16, 16)]
            t2 = qbuf_v[pl.ds((4 * g + 2) * 16, 16)]
            t3 = qbuf_v[pl.ds((4 * g + 3) * 16, 16)]
            u1 = jnp.where(m2, t0, t1)
            u2 = jnp.where(m2, t2, t3)
            v1 = u1 + _perm(u1, rot1)
            v2 = u2 + _perm(u2, rot1)
            f0 = jnp.where(m1, v1, v2)
            outc_v[pl.ds(c * _CH + g * 16, 16)] = _perm(f0, bitrev)
            return carry

        lax.fori_loop(0, _CH // 16, merge_body, 0)

    pltpu.sync_copy(outc_v, out_hbm.at[pl.ds(base, _BPW)])


def kernel(users, items, user_table, item_table, W_beta, b_beta):
    users_i = users.astype(jnp.int32)
    items_i = items.astype(jnp.int32)
    w = W_beta.reshape(_D)
    binit = jnp.pad(b_beta.reshape(1), (0, 15))

    mesh = plsc.VectorSubcoreMesh(core_axis_name="c", subcore_axis_name="s")
    f = pl.kernel(
        _gmf_body,
        mesh=mesh,
        out_type=jax.ShapeDtypeStruct((_B,), jnp.float32),
        scratch_types=[
            pltpu.VMEM((_BPW,), jnp.int32),
            pltpu.VMEM((_BPW,), jnp.int32),
            pltpu.VMEM((_CH, _D), jnp.float32),
            pltpu.VMEM((_CH, _D), jnp.float32),
            pltpu.VMEM((_CH, _D), jnp.float32),
            pltpu.VMEM((_CH, _D), jnp.float32),
            pltpu.VMEM((_CH, _D), jnp.float32),
            pltpu.VMEM((_CH, _D), jnp.float32),
            pltpu.VMEM((_CH, _D), jnp.float32),
            pltpu.VMEM((_CH, _D), jnp.float32),
            pltpu.VMEM((_D,), jnp.float32),
            pltpu.VMEM((16,), jnp.float32),
            pltpu.VMEM((_CH * 4,), jnp.float32),
            pltpu.VMEM((_BPW,), jnp.float32),
            pltpu.SemaphoreType.DMA,
            pltpu.SemaphoreType.DMA,
            pltpu.SemaphoreType.DMA,
            pltpu.SemaphoreType.DMA,
        ],
    )
    out = f(users_i, items_i, user_table, item_table, w, binit)
    return out.reshape(_B, 1)


# speedup vs baseline: 1.3880x; 1.1114x over previous
import jax
import jax.numpy as jnp
from jax import lax
from jax.experimental import pallas as pl
from jax.experimental.pallas import tpu as pltpu
from jax.experimental.pallas import tpu_sc as plsc

_B = 16384
_D = 128
_NC = 2
_NS = 16
_NW = _NC * _NS
_BPW = _B // _NW
_CH = 128
_NCHUNK = _BPW // _CH
_NBUF = 2


def _gmf_body(users_hbm, items_hbm, utab_hbm, itab_hbm, w_hbm, binit_hbm,
              out_hbm, uidx_v, iidx_v, ub0, ub1, ib0, ib1,
              w_v, binit_v, qbuf_v, outc_v, sem0, sem1):
    cid = lax.axis_index("c")
    sid = lax.axis_index("s")
    wid = sid * _NC + cid
    base = wid * _BPW

    pltpu.sync_copy(w_hbm, w_v)
    pltpu.sync_copy(binit_hbm, binit_v)
    pltpu.sync_copy(users_hbm.at[pl.ds(base, _BPW)], uidx_v)
    pltpu.sync_copy(items_hbm.at[pl.ds(base, _BPW)], iidx_v)
    b_init = binit_v[...]
    w_regs = [w_v[pl.ds(j * 16, 16)] for j in range(8)]

    lane = lax.iota(jnp.int32, 16)
    rot8 = (lane + 8) & 15
    rot4 = (lane & 8) | ((lane + 4) & 7)
    rot2 = (lane & 12) | ((lane + 2) & 3)
    rot1 = (lane & 14) | ((lane + 1) & 1)
    bitrev = (((lane & 1) << 3) | ((lane & 2) << 1)
              | ((lane & 4) >> 1) | ((lane & 8) >> 3))
    m8 = lane < 8
    m4 = (lane & 4) == 0
    m2 = (lane & 2) == 0
    m1 = (lane & 1) == 0

    _dnums = lax.GatherDimensionNumbers(
        offset_dims=(), collapsed_slice_dims=(0,), start_index_map=(0,))

    def _perm(x, idx):
        return lax.gather(x, idx[:, None], _dnums, (1,),
                          mode=lax.GatherScatterMode.PROMISE_IN_BOUNDS)

    ubufs = (ub0, ub1)
    ibufs = (ib0, ib1)
    sems = (sem0, sem1)
    pending = [None] * _NBUF

    def start(c):
        k = c % _NBUF
        cu = pltpu.async_copy(
            utab_hbm.at[uidx_v.at[pl.ds(c * _CH, _CH)]], ubufs[k], sems[k])
        ci = pltpu.async_copy(
            itab_hbm.at[iidx_v.at[pl.ds(c * _CH, _CH)]], ibufs[k], sems[k])
        pending[k] = (cu, ci)

    for c in range(_NBUF - 1):
        start(c)

    for c in range(_NCHUNK):
        if c + _NBUF - 1 < _NCHUNK:
            start(c + _NBUF - 1)
        k = c % _NBUF
        cu, ci = pending[k]
        cu.wait()
        ci.wait()
        urows_v = ubufs[k]
        irows_v = ibufs[k]

        def row_acc(r):
            acc = b_init
            for j in range(8):
                acc = acc + (urows_v[r, pl.ds(j * 16, 16)]
                             * irows_v[r, pl.ds(j * 16, 16)]
                             * w_regs[j])
            return acc

        def quad_body(i, carry):
            a = row_acc(4 * i)
            b = row_acc(4 * i + 1)
            cc = row_acc(4 * i + 2)
            d = row_acc(4 * i + 3)
            p1 = jnp.where(m8, a + _perm(a, rot8), b + _perm(b, rot8))
            p2 = jnp.where(m8, cc + _perm(cc, rot8), d + _perm(d, rot8))
            q1 = p1 + _perm(p1, rot4)
            q2 = p2 + _perm(p2, rot4)
            s = jnp.where(m4, q1, q2)
            qbuf_v[pl.ds(i * 16, 16)] = s + _perm(s, rot2)
            return carry

        lax.fori_loop(0, _CH // 4, quad_body, 0)

        def merge_body(g, carry):
            t0 = qbuf_v[pl.ds((4 * g) * 16, 16)]
            t1 = qbuf_v[pl.ds((4 * g + 1) * 16, 16)]
            t2 = qbuf_v[pl.ds((4 * g + 2) * 16, 16)]
            t3 = qbuf_v[pl.ds((4 * g + 3) * 16, 16)]
            u1 = jnp.where(m2, t0, t1)
            u2 = jnp.where(m2, t2, t3)
            v1 = u1 + _perm(u1, rot1)
            v2 = u2 + _perm(u2, rot1)
            f0 = jnp.where(m1, v1, v2)
            outc_v[pl.ds(c * _CH + g * 16, 16)] = _perm(f0, bitrev)
            return carry

        lax.fori_loop(0, _CH // 16, merge_body, 0)

    pltpu.sync_copy(outc_v, out_hbm.at[pl.ds(base, _BPW)])


def kernel(users, items, user_table, item_table, W_beta, b_beta):
    users_i = users.astype(jnp.int32)
    items_i = items.astype(jnp.int32)
    w = W_beta.reshape(_D)
    binit = jnp.pad(b_beta.reshape(1), (0, 15))

    mesh = plsc.VectorSubcoreMesh(core_axis_name="c", subcore_axis_name="s")
    f = pl.kernel(
        _gmf_body,
        mesh=mesh,
        out_type=jax.ShapeDtypeStruct((_B,), jnp.float32),
        scratch_types=[
            pltpu.VMEM((_BPW,), jnp.int32),
            pltpu.VMEM((_BPW,), jnp.int32),
            pltpu.VMEM((_CH, _D), jnp.float32),
            pltpu.VMEM((_CH, _D), jnp.float32),
            pltpu.VMEM((_CH, _D), jnp.float32),
            pltpu.VMEM((_CH, _D), jnp.float32),
            pltpu.VMEM((_D,), jnp.float32),
            pltpu.VMEM((16,), jnp.float32),
            pltpu.VMEM((_CH * 4,), jnp.float32),
            pltpu.VMEM((_BPW,), jnp.float32),
            pltpu.SemaphoreType.DMA,
            pltpu.SemaphoreType.DMA,
        ],
    )
    out = f(users_i, items_i, user_table, item_table, w, binit)
    return out.reshape(_B, 1)


# speedup vs baseline: 1.4180x; 1.0216x over previous
import jax
import jax.numpy as jnp
from jax import lax
from jax.experimental import pallas as pl
from jax.experimental.pallas import tpu as pltpu
from jax.experimental.pallas import tpu_sc as plsc

_B = 16384
_D = 128
_NC = 2
_NS = 16
_NW = _NC * _NS
_BPW = _B // _NW
_CH = 128
_NCHUNK = _BPW // _CH
_NBUF = 2


def _gmf_body(users_hbm, items_hbm, utab_hbm, itab_hbm, w_hbm, b_hbm,
              out_hbm, uidx_v, iidx_v, ub0, ub1, ib0, ib1,
              w_v, binit_v, qbuf_v, outc_v, sem0, sem1, semp):
    cid = lax.axis_index("c")
    sid = lax.axis_index("s")
    wid = sid * _NC + cid
    base = wid * _BPW

    binit_v[pl.ds(1, 16)] = jnp.zeros((16,), jnp.float32)
    cw = pltpu.async_copy(w_hbm, w_v, semp)
    cb = pltpu.async_copy(b_hbm, binit_v.at[pl.ds(0, 1)], semp)
    cui = pltpu.async_copy(users_hbm.at[pl.ds(base, _BPW)], uidx_v, semp)
    cii = pltpu.async_copy(items_hbm.at[pl.ds(base, _BPW)], iidx_v, semp)
    cw.wait()
    cb.wait()
    cui.wait()
    cii.wait()
    b_init = binit_v[pl.ds(0, 16)]
    w_regs = [w_v[pl.ds(j * 16, 16)] for j in range(8)]

    lane = lax.iota(jnp.int32, 16)
    rot8 = (lane + 8) & 15
    rot4 = (lane & 8) | ((lane + 4) & 7)
    rot2 = (lane & 12) | ((lane + 2) & 3)
    rot1 = (lane & 14) | ((lane + 1) & 1)
    bitrev = (((lane & 1) << 3) | ((lane & 2) << 1)
              | ((lane & 4) >> 1) | ((lane & 8) >> 3))
    m8 = lane < 8
    m4 = (lane & 4) == 0
    m2 = (lane & 2) == 0
    m1 = (lane & 1) == 0

    _dnums = lax.GatherDimensionNumbers(
        offset_dims=(), collapsed_slice_dims=(0,), start_index_map=(0,))

    def _perm(x, idx):
        return lax.gather(x, idx[:, None], _dnums, (1,),
                          mode=lax.GatherScatterMode.PROMISE_IN_BOUNDS)

    ubufs = (ub0, ub1)
    ibufs = (ib0, ib1)
    sems = (sem0, sem1)
    pending = [None] * _NBUF

    def start(c):
        k = c % _NBUF
        cu = pltpu.async_copy(
            utab_hbm.at[uidx_v.at[pl.ds(c * _CH, _CH)]], ubufs[k], sems[k])
        ci = pltpu.async_copy(
            itab_hbm.at[iidx_v.at[pl.ds(c * _CH, _CH)]], ibufs[k], sems[k])
        pending[k] = (cu, ci)

    for c in range(_NBUF - 1):
        start(c)

    for c in range(_NCHUNK):
        if c + _NBUF - 1 < _NCHUNK:
            start(c + _NBUF - 1)
        k = c % _NBUF
        cu, ci = pending[k]
        cu.wait()
        ci.wait()
        urows_v = ubufs[k]
        irows_v = ibufs[k]

        def row_acc(r):
            acc = b_init
            for j in range(8):
                acc = acc + (urows_v[r, pl.ds(j * 16, 16)]
                             * irows_v[r, pl.ds(j * 16, 16)]
                             * w_regs[j])
            return acc

        def quad_body(i, carry):
            a = row_acc(4 * i)
            b = row_acc(4 * i + 1)
            cc = row_acc(4 * i + 2)
            d = row_acc(4 * i + 3)
            p1 = jnp.where(m8, a + _perm(a, rot8), b + _perm(b, rot8))
            p2 = jnp.where(m8, cc + _perm(cc, rot8), d + _perm(d, rot8))
            q1 = p1 + _perm(p1, rot4)
            q2 = p2 + _perm(p2, rot4)
            s = jnp.where(m4, q1, q2)
            qbuf_v[pl.ds(i * 16, 16)] = s + _perm(s, rot2)
            return carry

        lax.fori_loop(0, _CH // 4, quad_body, 0)

        def merge_body(g, carry):
            t0 = qbuf_v[pl.ds((4 * g) * 16, 16)]
            t1 = qbuf_v[pl.ds((4 * g + 1) * 16, 16)]
            t2 = qbuf_v[pl.ds((4 * g + 2) * 16, 16)]
            t3 = qbuf_v[pl.ds((4 * g + 3) * 16, 16)]
            u1 = jnp.where(m2, t0, t1)
            u2 = jnp.where(m2, t2, t3)
            v1 = u1 + _perm(u1, rot1)
            v2 = u2 + _perm(u2, rot1)
            f0 = jnp.where(m1, v1, v2)
            outc_v[pl.ds(c * _CH + g * 16, 16)] = _perm(f0, bitrev)
            return carry

        lax.fori_loop(0, _CH // 16, merge_body, 0)

    pltpu.sync_copy(outc_v, out_hbm.at[pl.ds(base, _BPW)])


def kernel(users, items, user_table, item_table, W_beta, b_beta):
    users_i = users.astype(jnp.int32)
    items_i = items.astype(jnp.int32)
    w = W_beta.reshape(_D)

    mesh = plsc.VectorSubcoreMesh(core_axis_name="c", subcore_axis_name="s")
    f = pl.kernel(
        _gmf_body,
        mesh=mesh,
        out_type=jax.ShapeDtypeStruct((_B,), jnp.float32),
        scratch_types=[
            pltpu.VMEM((_BPW,), jnp.int32),
            pltpu.VMEM((_BPW,), jnp.int32),
            pltpu.VMEM((_CH, _D), jnp.float32),
            pltpu.VMEM((_CH, _D), jnp.float32),
            pltpu.VMEM((_CH, _D), jnp.float32),
            pltpu.VMEM((_CH, _D), jnp.float32),
            pltpu.VMEM((_D,), jnp.float32),
            pltpu.VMEM((17,), jnp.float32),
            pltpu.VMEM((_CH * 4,), jnp.float32),
            pltpu.VMEM((_BPW,), jnp.float32),
            pltpu.SemaphoreType.DMA,
            pltpu.SemaphoreType.DMA,
            pltpu.SemaphoreType.DMA,
        ],
    )
    out = f(users_i, items_i, user_table, item_table, w, b_beta)
    return out.reshape(_B, 1)
